# baseline (speedup 1.0000x reference)
import jax
import jax.numpy as jnp
from jax import lax
from jax.experimental import pallas as pl
from jax.experimental.pallas import tpu as pltpu
from jax.experimental.pallas import tpu_sc as plsc

NC = 2
NS = 16

BATCH = 4096
HIST = 200
VOCAB = 1000
D = 64
N_IDX = BATCH * HIST
N_PER_SC = N_IDX // NC

R_ROWS = 6400
N_ROUNDS = N_PER_SC // R_ROWS
T_ROWS = R_ROWS // NS
NBUF = 3


def _body(tokens_hbm, table_hbm, out_hbm, idx_v, local_v, shared,
          isem, gsem, csem, wsem):
    c = lax.axis_index("c")
    s = lax.axis_index("s")

    def idxload(r):
        return pltpu.make_async_copy(
            tokens_hbm.at[c, r, s],
            idx_v.at[lax.rem(r, 3)],
            isem.at[lax.rem(r, 3)],
        )

    def gather(r):
        return pltpu.make_async_copy(
            table_hbm.at[idx_v.at[lax.rem(r, 3)]],
            local_v.at[lax.rem(r, 2)],
            gsem.at[lax.rem(r, 2)],
        )

    def copy(r):
        return pltpu.make_async_copy(
            local_v.at[lax.rem(r, 2)],
            shared.at[lax.rem(r, NBUF), pl.ds(s * T_ROWS, T_ROWS)],
            csem,
        )

    def write(r):
        return pltpu.make_async_copy(
            shared.at[lax.rem(r, NBUF)],
            out_hbm.at[pl.ds((c * N_ROUNDS + r) * R_ROWS, R_ROWS)],
            wsem,
        )

    idxload(0).start()
    idxload(1).start()
    idxload(2).start()
    idxload(0).wait()
    gather(0).start()
    idxload(1).wait()
    gather(1).start()
    gather(0).wait()
    copy(0).start()

    @pl.loop(0, N_ROUNDS)
    def _round(r):
        @pl.when(jnp.logical_and(r >= NBUF - 1,
                                 s == lax.rem(r - (NBUF - 1), NS)))
        def _():
            write(r - (NBUF - 1)).wait()

        plsc.subcore_barrier()

        @pl.when(r + 3 < N_ROUNDS)
        def _():
            idxload(r + 3).start()

        copy(r).wait()

        @pl.when(r + 2 < N_ROUNDS)
        def _():
            idxload(r + 2).wait()
            gather(r + 2).start()

        @pl.when(r + 1 < N_ROUNDS)
        def _():
            gather(r + 1).wait()
            copy(r + 1).start()

        plsc.subcore_barrier()

        @pl.when(s == lax.rem(r, NS))
        def _():
            write(r).start()

    for r in range(N_ROUNDS - (NBUF - 1), N_ROUNDS):
        @pl.when(s == lax.rem(jnp.int32(r), NS))
        def _():
            write(r).wait()


def kernel(tokens, table):
    idx4 = tokens.reshape(NC, N_ROUNDS, NS, T_ROWS)
    mesh = plsc.VectorSubcoreMesh(core_axis_name="c", subcore_axis_name="s")
    out = pl.kernel(
        _body,
        out_type=jax.ShapeDtypeStruct((N_IDX, D), jnp.float32),
        mesh=mesh,
        compiler_params=pltpu.CompilerParams(use_tc_tiling_on_sc=False),
        scratch_types=[
            pltpu.VMEM((3, T_ROWS), jnp.int32),
            pltpu.VMEM((2, T_ROWS, D), jnp.float32),
            pltpu.VMEM_SHARED((NBUF, R_ROWS, D), jnp.float32),
            pltpu.SemaphoreType.DMA((3,)),
            pltpu.SemaphoreType.DMA((2,)),
            pltpu.SemaphoreType.DMA,
            pltpu.SemaphoreType.DMA,
        ],
    )(idx4, table)
    return out.reshape(BATCH, HIST, D)


# speedup vs baseline: 1.0017x; 1.0017x over previous
import jax
import jax.numpy as jnp
from jax import lax
from jax.experimental import pallas as pl
from jax.experimental.pallas import tpu as pltpu
from jax.experimental.pallas import tpu_sc as plsc

NC = 2
NS = 16

BATCH = 4096
HIST = 200
VOCAB = 1000
D = 64
N_IDX = BATCH * HIST
N_PER_SC = N_IDX // NC

R_ROWS = 6400
N_ROUNDS = N_PER_SC // R_ROWS
T_ROWS = R_ROWS // NS
NBUF = 3


def _body(tokens_hbm, table_hbm, out_hbm, idx_v, local_v, shared,
          isem, gsem, csem, wsem):
    c = lax.axis_index("c")
    s = lax.axis_index("s")

    def idxload(r):
        return pltpu.make_async_copy(
            tokens_hbm.at[c, r, s],
            idx_v.at[lax.rem(r, 3)],
            isem.at[lax.rem(r, 3)],
        )

    def gather(r):
        return pltpu.make_async_copy(
            table_hbm.at[idx_v.at[lax.rem(r, 3)]],
            local_v.at[lax.rem(r, 2)],
            gsem.at[lax.rem(r, 2)],
        )

    def copy(r):
        return pltpu.make_async_copy(
            local_v.at[lax.rem(r, 2)],
            shared.at[lax.rem(r, NBUF), pl.ds(s * T_ROWS, T_ROWS)],
            csem,
        )

    def write(r):
        return pltpu.make_async_copy(
            shared.at[lax.rem(r, NBUF)],
            out_hbm.at[pl.ds((c * N_ROUNDS + r) * R_ROWS, R_ROWS)],
            wsem,
        )

    idxload(0).start()
    idxload(1).start()
    idxload(2).start()
    idxload(0).wait()
    gather(0).start()
    idxload(1).wait()
    gather(1).start()
    gather(0).wait()
    copy(0).start()

    @pl.loop(0, N_ROUNDS)
    def _round(r):
        @pl.when(jnp.logical_and(r >= NBUF - 1,
                                 s == lax.rem(r - (NBUF - 1), NS)))
        def _():
            write(r - (NBUF - 1)).wait()

        copy(r).wait()
        plsc.subcore_barrier()

        @pl.when(s == lax.rem(r, NS))
        def _():
            write(r).start()

        @pl.when(r + 3 < N_ROUNDS)
        def _():
            idxload(r + 3).start()

        @pl.when(r + 2 < N_ROUNDS)
        def _():
            idxload(r + 2).wait()
            gather(r + 2).start()

        @pl.when(r + 1 < N_ROUNDS)
        def _():
            gather(r + 1).wait()
            copy(r + 1).start()

    for r in range(N_ROUNDS - (NBUF - 1), N_ROUNDS):
        @pl.when(s == lax.rem(jnp.int32(r), NS))
        def _():
            write(r).wait()


def kernel(tokens, table):
    idx4 = tokens.reshape(NC, N_ROUNDS, NS, T_ROWS)
    mesh = plsc.VectorSubcoreMesh(core_axis_name="c", subcore_axis_name="s")
    out = pl.kernel(
        _body,
        out_type=jax.ShapeDtypeStruct((N_IDX, D), jnp.float32),
        mesh=mesh,
        compiler_params=pltpu.CompilerParams(use_tc_tiling_on_sc=False),
        scratch_types=[
            pltpu.VMEM((3, T_ROWS), jnp.int32),
            pltpu.VMEM((2, T_ROWS, D), jnp.float32),
            pltpu.VMEM_SHARED((NBUF, R_ROWS, D), jnp.float32),
            pltpu.SemaphoreType.DMA((3,)),
            pltpu.SemaphoreType.DMA((2,)),
            pltpu.SemaphoreType.DMA,
            pltpu.SemaphoreType.DMA,
        ],
    )(idx4, table)
    return out.reshape(BATCH, HIST, D)
